# baseline (speedup 1.0000x reference)
import functools

import jax
import jax.numpy as jnp
from jax.experimental import pallas as pl
from jax.experimental.pallas import tpu as pltpu

NE = 8
TK = 2
D = 2048
F = 4096
T = 128
NT = 39
PAD = NT * T
NC1 = 2
FC = F // NC1
NC2 = 2
NCH = D // NC2
S1 = NC1 * NT
S2 = NC2 * NT
NJ1 = NC1 * NE + 2
NJ2 = NC2 * NE + 2


def _mk_sched(tile_go, tile_expert, nchunk):
    s = nchunk * NT
    te_step = jnp.tile(tile_expert, nchunk)
    c_step = jnp.repeat(jnp.arange(nchunk, dtype=jnp.int32), NT)
    prev_te = jnp.concatenate([-jnp.ones((1,), jnp.int32), te_step[:-1]])
    t_idx = jnp.tile(jnp.arange(NT, dtype=jnp.int32), nchunk)
    first = ((t_idx == 0) | (te_step != prev_te)).astype(jnp.int32)
    j = jnp.cumsum(first) - 1
    buf = (j % 2).astype(jnp.int32)
    njmax = nchunk * NE + 2
    chunk_e = (-jnp.ones((njmax,), jnp.int32)).at[j].set(te_step)
    chunk_c = (-jnp.ones((njmax,), jnp.int32)).at[j].set(c_step)
    ie = jnp.where(first == 1, chunk_e[jnp.minimum(j + 1, njmax - 1)], -1).astype(jnp.int32)
    ic = jnp.where(first == 1, chunk_c[jnp.minimum(j + 1, njmax - 1)], -1).astype(jnp.int32)
    pr_e = chunk_e[:2]
    pr_c = chunk_c[:2]
    return jnp.concatenate([tile_go, first, buf, ie, ic, pr_e, pr_c]).astype(jnp.int32)


def _ffn1_body(sc_ref, xs_ref, w1_hbm, sw_ref, h_ref, wbuf, sems):
    c = pl.program_id(0)
    t = pl.program_id(1)
    s = c * NT + t

    @pl.when(s == 0)
    def _prime():
        for r in range(2):
            e0 = sc_ref[NT + 4 * S1 + r]
            c0 = sc_ref[NT + 4 * S1 + 2 + r]

            @pl.when(e0 >= 0)
            def _():
                pltpu.make_async_copy(
                    w1_hbm.at[e0, :, pl.ds(c0 * FC, FC)], wbuf.at[r], sems.at[r]
                ).start()

    first = sc_ref[NT + s]

    @pl.when((first == 1) & (s > 0))
    def _issue_next():
        e2 = sc_ref[NT + 2 * S1 + s]
        c2 = sc_ref[NT + 3 * S1 + s]

        @pl.when(e2 >= 0)
        def _():
            b2 = 1 - sc_ref[NT + S1 + s]
            pltpu.make_async_copy(
                w1_hbm.at[e2, :, pl.ds(c2 * FC, FC)], wbuf.at[b2], sems.at[b2]
            ).start()

    @pl.when(first == 1)
    def _wait_cur():
        b = sc_ref[NT + S1 + s]
        pltpu.make_async_copy(
            w1_hbm.at[0, :, pl.ds(0, FC)], wbuf.at[b], sems.at[b]
        ).wait()

    @pl.when(sc_ref[t] == 1)
    def _compute():
        b = sc_ref[NT + S1 + s]
        xb = xs_ref[...]
        acc = jnp.dot(xb, wbuf[b], preferred_element_type=jnp.float32)
        h = jax.nn.gelu(acc) * sw_ref[...]
        h_ref[...] = h.astype(jnp.bfloat16)


def _ffn2_body(sc_ref, h_ref, w2_hbm, o_ref, wbuf, sems):
    c = pl.program_id(0)
    t = pl.program_id(1)
    s = c * NT + t

    @pl.when(s == 0)
    def _prime():
        for r in range(2):
            e0 = sc_ref[NT + 4 * S2 + r]
            c0 = sc_ref[NT + 4 * S2 + 2 + r]

            @pl.when(e0 >= 0)
            def _():
                pltpu.make_async_copy(
                    w2_hbm.at[e0, :, pl.ds(c0 * NCH, NCH)], wbuf.at[r], sems.at[r]
                ).start()

    first = sc_ref[NT + s]

    @pl.when((first == 1) & (s > 0))
    def _issue_next():
        e2 = sc_ref[NT + 2 * S2 + s]
        c2 = sc_ref[NT + 3 * S2 + s]

        @pl.when(e2 >= 0)
        def _():
            b2 = 1 - sc_ref[NT + S2 + s]
            pltpu.make_async_copy(
                w2_hbm.at[e2, :, pl.ds(c2 * NCH, NCH)], wbuf.at[b2], sems.at[b2]
            ).start()

    @pl.when(first == 1)
    def _wait_cur():
        b = sc_ref[NT + S2 + s]
        pltpu.make_async_copy(
            w2_hbm.at[0, :, pl.ds(0, NCH)], wbuf.at[b], sems.at[b]
        ).wait()

    @pl.when(sc_ref[t] == 1)
    def _compute():
        b = sc_ref[NT + S2 + s]
        hb = h_ref[...].astype(jnp.float32)
        o_ref[...] = jnp.dot(hb, wbuf[b], preferred_element_type=jnp.float32)


def _grouped_ffn(sc1, sc2, xs, sw, w1, w2):
    h = pl.pallas_call(
        _ffn1_body,
        grid_spec=pltpu.PrefetchScalarGridSpec(
            num_scalar_prefetch=1,
            grid=(NC1, NT),
            in_specs=[
                pl.BlockSpec((T, D), lambda c, t, sc: (t, 0)),
                pl.BlockSpec(memory_space=pl.ANY),
                pl.BlockSpec((T, 1), lambda c, t, sc: (t, 0)),
            ],
            out_specs=pl.BlockSpec((T, FC), lambda c, t, sc: (t, c)),
            scratch_shapes=[
                pltpu.VMEM((2, D, FC), jnp.float32),
                pltpu.SemaphoreType.DMA((2,)),
            ],
        ),
        out_shape=jax.ShapeDtypeStruct((PAD, F), jnp.bfloat16),
    )(sc1, xs, w1, sw)
    out = pl.pallas_call(
        _ffn2_body,
        grid_spec=pltpu.PrefetchScalarGridSpec(
            num_scalar_prefetch=1,
            grid=(NC2, NT),
            in_specs=[
                pl.BlockSpec((T, F), lambda n, t, sc: (t, 0)),
                pl.BlockSpec(memory_space=pl.ANY),
            ],
            out_specs=pl.BlockSpec((T, NCH), lambda n, t, sc: (t, n)),
            scratch_shapes=[
                pltpu.VMEM((2, F, NCH), jnp.float32),
                pltpu.SemaphoreType.DMA((2,)),
            ],
        ),
        out_shape=jax.ShapeDtypeStruct((PAD, D), jnp.float32),
    )(sc2, h, w2)
    return out


def kernel(x, expert_weights, expert_indices, scores, w1, w2):
    sl, bs, hs = x.shape
    ntok = sl * bs
    xf = x.reshape(ntok, hs)
    ei = expert_indices.reshape(-1).astype(jnp.int32)
    ew = expert_weights.reshape(-1)

    oh = (ei[:, None] == jnp.arange(NE, dtype=jnp.int32)[None, :]).astype(jnp.int32)
    hist = oh.sum(axis=0)
    rank = jnp.take_along_axis(jnp.cumsum(oh, axis=0) - 1, ei[:, None], axis=1)[:, 0]
    padded = ((hist + T - 1) // T) * T
    pend = jnp.cumsum(padded)
    poff = pend - padded
    pos = poff[ei] + rank
    slot_token = jnp.zeros((PAD,), jnp.int32).at[pos].set(
        jnp.arange(ntok * TK, dtype=jnp.int32) // TK)
    slot_w = jnp.zeros((PAD,), jnp.float32).at[pos].set(ew)
    tT = jnp.arange(NT, dtype=jnp.int32) * T
    te_raw = jnp.searchsorted(pend, tT, side="right").astype(jnp.int32)
    last_e = jnp.max(jnp.where(hist > 0, jnp.arange(NE, dtype=jnp.int32), -1))
    tile_expert = jnp.minimum(te_raw, last_e)
    nreal = poff[tile_expert] + hist[tile_expert] - tT
    tile_go = (nreal > 0).astype(jnp.int32)
    sc1 = _mk_sched(tile_go, tile_expert, NC1)
    sc2 = _mk_sched(tile_go, tile_expert, NC2)

    xs = jnp.take(xf, slot_token, axis=0)

    os_ = _grouped_ffn(sc1, sc2, xs, slot_w[:, None], w1, w2)

    y = jnp.take(os_, pos, axis=0).reshape(ntok, TK, hs).sum(axis=1)
    return y.reshape(sl, bs, hs)
